# Initial kernel scaffold; baseline (speedup 1.0000x reference)
import jax
import jax.numpy as jnp
from jax.experimental import pallas as pl


def kernel(x, edge_index, W1, b1, W2, b2, Wfc, bfc):
    raise NotImplementedError("write your pallas kernel here")


# speedup vs baseline: 12.9483x; 12.9483x over previous
import functools

import jax
import jax.numpy as jnp
from jax import lax
from jax.experimental import pallas as pl
from jax.experimental.pallas import tpu as pltpu
from jax.experimental.pallas import tpu_sc as plsc

N = 10000
NPAD = 10240
E = 320000
D = 128
NC = 2
NS = 16
NW = NC * NS
EPW = E // NW
K = 80
NCHUNK = EPW // K
RPT = 624
RTAIL = N - NS * RPT

_mesh = plsc.VectorSubcoreMesh(core_axis_name="c", subcore_axis_name="s")


def _copy_rows(src_ref, dst_ref, s):
    pltpu.sync_copy(src_ref.at[pl.ds(s * RPT, RPT)], dst_ref.at[pl.ds(s * RPT, RPT)])

    @pl.when(s == NS - 1)
    def _():
        pltpu.sync_copy(src_ref.at[pl.ds(NS * RPT, RTAIL)],
                        dst_ref.at[pl.ds(NS * RPT, RTAIL)])


@functools.partial(
    pl.kernel,
    out_type=jax.ShapeDtypeStruct((NW * NPAD,), jnp.float32),
    mesh=_mesh,
    scratch_types=[
        pltpu.VMEM((NPAD,), jnp.float32),
        pltpu.VMEM((K,), jnp.int32),
    ],
    compiler_params=pltpu.CompilerParams(needs_layout_passes=False),
)
def _sc_degree(dst_hbm, out_hbm, hist_v, dst_v):
    c = lax.axis_index("c")
    s = lax.axis_index("s")
    wid = c * NS + s
    zeros16 = jnp.zeros((16,), jnp.float32)
    ones16 = jnp.ones((16,), jnp.float32)

    def zero(j, carry):
        hist_v[pl.ds(j * 16, 16)] = zeros16
        return carry

    lax.fori_loop(0, NPAD // 16, zero, 0)

    def chunk(i, carry):
        base = wid * EPW + i * K
        pltpu.sync_copy(dst_hbm.at[pl.ds(base, K)], dst_v)
        for j in range(K // 16):
            vec = dst_v[pl.ds(j * 16, 16)]
            plsc.addupdate_scatter(hist_v, [vec], ones16)
        return carry

    lax.fori_loop(0, NCHUNK, chunk, 0)
    pltpu.sync_copy(hist_v, out_hbm.at[pl.ds(wid * NPAD, NPAD)])


@functools.partial(
    pl.kernel,
    out_type=jax.ShapeDtypeStruct((NC, N, D), jnp.float32),
    mesh=_mesh,
    scratch_types=[
        pltpu.VMEM_SHARED((N, D), jnp.float32),
        pltpu.VMEM((K,), jnp.int32),
        pltpu.VMEM((K,), jnp.int32),
        pltpu.VMEM((K, D), jnp.float32),
        pltpu.SemaphoreType.DMA,
    ],
)
def _sc_scatter(g_hbm, src_hbm, dst_hbm, zeros_hbm, out_hbm,
                acc, src_v, dst_v, rows_v, sem):
    c = lax.axis_index("c")
    s = lax.axis_index("s")
    wid = c * NS + s
    _copy_rows(zeros_hbm, acc, s)
    plsc.subcore_barrier()

    def chunk(i, carry):
        base = wid * EPW + i * K
        pltpu.sync_copy(src_hbm.at[pl.ds(base, K)], src_v)
        pltpu.sync_copy(dst_hbm.at[pl.ds(base, K)], dst_v)
        pltpu.async_copy(g_hbm.at[src_v], rows_v, sem).wait()
        pltpu.sync_copy(rows_v, acc.at[dst_v], add=True)
        return carry

    lax.fori_loop(0, NCHUNK, chunk, 0)
    plsc.subcore_barrier()
    _copy_rows(acc, out_hbm.at[c], s)


RB = 1000


def _tc_dis_body(degs_ref, dis_ref):
    deg = jnp.sum(degs_ref[...], axis=0) + 1.0
    dis_ref[...] = 1.0 / jnp.sqrt(deg)


def _tc_g1_body(x_ref, w1_ref, dis_ref, g1_ref):
    h = jnp.dot(x_ref[...], w1_ref[...], preferred_element_type=jnp.float32,
                precision=lax.Precision.HIGHEST)
    g1_ref[...] = dis_ref[...] * h


def _tc_layer2_body(p_ref, g1_ref, dis_ref, b1_ref, w2_ref, g2_ref):
    dis = dis_ref[...]
    p = p_ref[...]
    s = p[0] + p[1] + g1_ref[...]
    h1 = jnp.maximum(dis * s + b1_ref[...], 0.0)
    h2 = jnp.dot(h1, w2_ref[...], preferred_element_type=jnp.float32,
                 precision=lax.Precision.HIGHEST)
    g2_ref[...] = dis * h2


def _tc_final_body(q_ref, g2_ref, dis_ref, b2_ref, wfc_ref, bfc_ref, out_ref):
    dis = dis_ref[...]
    q = q_ref[...]
    s = q[0] + q[1] + g2_ref[...]
    h2 = dis * s + b2_ref[...]
    out_ref[...] = jnp.dot(h2, wfc_ref[...], preferred_element_type=jnp.float32,
                           precision=lax.Precision.HIGHEST) + bfc_ref[...]


def _row_spec(cols):
    return pl.BlockSpec((RB, cols), lambda i: (i, 0))


def _full_spec(shape):
    return pl.BlockSpec(shape, lambda i: tuple(0 for _ in shape))


def _pair_spec(cols):
    return pl.BlockSpec((NC, RB, cols), lambda i: (0, i, 0))


def kernel(x, edge_index, W1, b1, W2, b2, Wfc, bfc):
    src = edge_index[0].astype(jnp.int32)
    dst = edge_index[1].astype(jnp.int32)
    zerosD = jnp.zeros((N, D), jnp.float32)
    b1r = b1.reshape(1, D)
    b2r = b2.reshape(1, D)
    bfcr = bfc.reshape(1, -1)
    ncls = Wfc.shape[1]

    degs = _sc_degree(dst).reshape(NW, NPAD // 128, 128)

    dis2d = pl.pallas_call(
        _tc_dis_body,
        in_specs=[pl.BlockSpec((NW, NPAD // 128, 128), lambda: (0, 0, 0))],
        out_specs=pl.BlockSpec((NPAD // 128, 128), lambda: (0, 0)),
        out_shape=jax.ShapeDtypeStruct((NPAD // 128, 128), jnp.float32),
    )(degs)
    dis = dis2d.reshape(NPAD, 1)[:N]

    g1 = pl.pallas_call(
        _tc_g1_body,
        grid=(N // RB,),
        in_specs=[_row_spec(D), _full_spec((D, D)), _row_spec(1)],
        out_specs=_row_spec(D),
        out_shape=jax.ShapeDtypeStruct((N, D), jnp.float32),
    )(x, W1, dis)

    p1 = _sc_scatter(g1, src, dst, zerosD)

    g2 = pl.pallas_call(
        _tc_layer2_body,
        grid=(N // RB,),
        in_specs=[_pair_spec(D), _row_spec(D), _row_spec(1),
                  _full_spec((1, D)), _full_spec((D, D))],
        out_specs=_row_spec(D),
        out_shape=jax.ShapeDtypeStruct((N, D), jnp.float32),
    )(p1, g1, dis, b1r, W2)

    p2 = _sc_scatter(g2, src, dst, zerosD)

    out = pl.pallas_call(
        _tc_final_body,
        grid=(N // RB,),
        in_specs=[_pair_spec(D), _row_spec(D), _row_spec(1),
                  _full_spec((1, D)), _full_spec((D, ncls)), _full_spec((1, ncls))],
        out_specs=_row_spec(ncls),
        out_shape=jax.ShapeDtypeStruct((N, ncls), jnp.float32),
    )(p2, g2, dis, b2r, Wfc, bfcr)

    return out


# speedup vs baseline: 34.5788x; 2.6705x over previous
import functools

import jax
import jax.numpy as jnp
from jax import lax
from jax.experimental import pallas as pl
from jax.experimental.pallas import tpu as pltpu
from jax.experimental.pallas import tpu_sc as plsc

N = 10000
NPAD = 10240
E = 320000
D = 128
NC = 2
NS = 16
NW = NC * NS
EPW = E // NW
K = 80
NCHUNK = EPW // K
RPT = 624
RTAIL = N - NS * RPT

_mesh = plsc.VectorSubcoreMesh(core_axis_name="c", subcore_axis_name="s")


def _copy_rows(src_ref, dst_ref, s):
    pltpu.sync_copy(src_ref.at[pl.ds(s * RPT, RPT)], dst_ref.at[pl.ds(s * RPT, RPT)])

    @pl.when(s == NS - 1)
    def _():
        pltpu.sync_copy(src_ref.at[pl.ds(NS * RPT, RTAIL)],
                        dst_ref.at[pl.ds(NS * RPT, RTAIL)])


@functools.partial(
    pl.kernel,
    out_type=jax.ShapeDtypeStruct((NW * NPAD,), jnp.float32),
    mesh=_mesh,
    scratch_types=[
        pltpu.VMEM((NPAD,), jnp.float32),
        pltpu.VMEM((EPW,), jnp.int32),
    ],
    compiler_params=pltpu.CompilerParams(needs_layout_passes=False),
)
def _sc_degree(dst_hbm, out_hbm, hist_v, dst_v):
    c = lax.axis_index("c")
    s = lax.axis_index("s")
    wid = c * NS + s
    zeros16 = jnp.zeros((16,), jnp.float32)
    ones16 = jnp.ones((16,), jnp.float32)

    def zero(j, carry):
        hist_v[pl.ds(j * 16, 16)] = zeros16
        return carry

    lax.fori_loop(0, NPAD // 16, zero, 0)
    pltpu.sync_copy(dst_hbm.at[pl.ds(wid * EPW, EPW)], dst_v)

    def step(j, carry):
        vec = dst_v[pl.ds(j * 16, 16)]
        plsc.addupdate_scatter(hist_v, [vec], ones16)
        return carry

    lax.fori_loop(0, EPW // 16, step, 0)
    pltpu.sync_copy(hist_v, out_hbm.at[pl.ds(wid * NPAD, NPAD)])


NBUF = 4
GMAX = (NCHUNK - 1) // NBUF


@functools.partial(
    pl.kernel,
    out_type=jax.ShapeDtypeStruct((NC, N, D), jnp.float32),
    mesh=_mesh,
    scratch_types=[
        pltpu.VMEM_SHARED((N, D), jnp.float32),
        pltpu.VMEM((NBUF, K), jnp.int32),
        pltpu.VMEM((NBUF, K), jnp.int32),
        pltpu.VMEM((NBUF, K, D), jnp.float32),
        pltpu.SemaphoreType.DMA((NBUF,)),
        pltpu.SemaphoreType.DMA((NBUF,)),
        pltpu.SemaphoreType.DMA((NBUF,)),
    ],
)
def _sc_scatter(g_hbm, src_hbm, dst_hbm, zeros_hbm, out_hbm,
                acc, src_v, dst_v, rows_v, semg, semis, semid):
    c = lax.axis_index("c")
    s = lax.axis_index("s")
    wid = c * NS + s
    _copy_rows(zeros_hbm, acc, s)

    def fire_src(b, i):
        pltpu.async_copy(src_hbm.at[pl.ds(wid * EPW + i * K, K)],
                         src_v.at[b], semis.at[b])

    def fire_dst(b, i):
        pltpu.async_copy(dst_hbm.at[pl.ds(wid * EPW + i * K, K)],
                         dst_v.at[b], semid.at[b])

    def fire_gather(b):
        pltpu.async_copy(g_hbm.at[src_v.at[b]], rows_v.at[b], semg.at[b])

    for b in range(NBUF):
        fire_src(b, b)
        fire_dst(b, b)
    for b in range(NBUF):
        pltpu.make_async_copy(src_hbm.at[pl.ds(0, K)], src_v.at[b],
                              semis.at[b]).wait()
        fire_gather(b)

    plsc.subcore_barrier()

    def group(g, carry):
        has_next = g < GMAX - 1
        for b in range(NBUF):
            i = g * NBUF + b
            pltpu.make_async_copy(g_hbm.at[src_v.at[b]], rows_v.at[b],
                                  semg.at[b]).wait()

            @pl.when(has_next)
            def _():
                fire_src(b, i + NBUF)
            pltpu.make_async_copy(dst_hbm.at[pl.ds(0, K)], dst_v.at[b],
                                  semid.at[b]).wait()
            pltpu.sync_copy(rows_v.at[b], acc.at[dst_v.at[b]], add=True)

            @pl.when(has_next)
            def _():
                fire_dst(b, i + NBUF)
                pltpu.make_async_copy(src_hbm.at[pl.ds(0, K)], src_v.at[b],
                                      semis.at[b]).wait()
                fire_gather(b)
        return carry

    lax.fori_loop(0, GMAX, group, 0)

    tail = NCHUNK - 1
    pltpu.sync_copy(src_hbm.at[pl.ds(wid * EPW + tail * K, K)], src_v.at[0])
    pltpu.sync_copy(dst_hbm.at[pl.ds(wid * EPW + tail * K, K)], dst_v.at[0])
    pltpu.async_copy(g_hbm.at[src_v.at[0]], rows_v.at[0], semg.at[0])
    pltpu.make_async_copy(g_hbm.at[src_v.at[0]], rows_v.at[0], semg.at[0]).wait()
    pltpu.sync_copy(rows_v.at[0], acc.at[dst_v.at[0]], add=True)

    plsc.subcore_barrier()
    _copy_rows(acc, out_hbm.at[c], s)


RB = 1000


def _tc_dis_body(degs_ref, dis_ref):
    deg = jnp.sum(degs_ref[...], axis=0) + 1.0
    dis_ref[...] = 1.0 / jnp.sqrt(deg)


def _tc_g1_body(x_ref, w1_ref, dis_ref, g1_ref):
    h = jnp.dot(x_ref[...], w1_ref[...], preferred_element_type=jnp.float32,
                precision=lax.Precision.HIGHEST)
    g1_ref[...] = dis_ref[...] * h


def _tc_layer2_body(p_ref, g1_ref, dis_ref, b1_ref, w2_ref, g2_ref):
    dis = dis_ref[...]
    p = p_ref[...]
    s = p[0] + p[1] + g1_ref[...]
    h1 = jnp.maximum(dis * s + b1_ref[...], 0.0)
    h2 = jnp.dot(h1, w2_ref[...], preferred_element_type=jnp.float32,
                 precision=lax.Precision.HIGHEST)
    g2_ref[...] = dis * h2


def _tc_final_body(q_ref, g2_ref, dis_ref, b2_ref, wfc_ref, bfc_ref, out_ref):
    dis = dis_ref[...]
    q = q_ref[...]
    s = q[0] + q[1] + g2_ref[...]
    h2 = dis * s + b2_ref[...]
    out_ref[...] = jnp.dot(h2, wfc_ref[...], preferred_element_type=jnp.float32,
                           precision=lax.Precision.HIGHEST) + bfc_ref[...]


def _row_spec(cols):
    return pl.BlockSpec((RB, cols), lambda i: (i, 0))


def _full_spec(shape):
    return pl.BlockSpec(shape, lambda i: tuple(0 for _ in shape))


def _pair_spec(cols):
    return pl.BlockSpec((NC, RB, cols), lambda i: (0, i, 0))


def kernel(x, edge_index, W1, b1, W2, b2, Wfc, bfc):
    src = edge_index[0].astype(jnp.int32)
    dst = edge_index[1].astype(jnp.int32)
    zerosD = jnp.zeros((N, D), jnp.float32)
    b1r = b1.reshape(1, D)
    b2r = b2.reshape(1, D)
    bfcr = bfc.reshape(1, -1)
    ncls = Wfc.shape[1]

    degs = _sc_degree(dst).reshape(NW, NPAD // 128, 128)

    dis2d = pl.pallas_call(
        _tc_dis_body,
        in_specs=[pl.BlockSpec((NW, NPAD // 128, 128), lambda: (0, 0, 0))],
        out_specs=pl.BlockSpec((NPAD // 128, 128), lambda: (0, 0)),
        out_shape=jax.ShapeDtypeStruct((NPAD // 128, 128), jnp.float32),
    )(degs)
    dis = dis2d.reshape(NPAD, 1)[:N]

    g1 = pl.pallas_call(
        _tc_g1_body,
        grid=(N // RB,),
        in_specs=[_row_spec(D), _full_spec((D, D)), _row_spec(1)],
        out_specs=_row_spec(D),
        out_shape=jax.ShapeDtypeStruct((N, D), jnp.float32),
    )(x, W1, dis)

    p1 = _sc_scatter(g1, src, dst, zerosD)

    g2 = pl.pallas_call(
        _tc_layer2_body,
        grid=(N // RB,),
        in_specs=[_pair_spec(D), _row_spec(D), _row_spec(1),
                  _full_spec((1, D)), _full_spec((D, D))],
        out_specs=_row_spec(D),
        out_shape=jax.ShapeDtypeStruct((N, D), jnp.float32),
    )(p1, g1, dis, b1r, W2)

    p2 = _sc_scatter(g2, src, dst, zerosD)

    out = pl.pallas_call(
        _tc_final_body,
        grid=(N // RB,),
        in_specs=[_pair_spec(D), _row_spec(D), _row_spec(1),
                  _full_spec((1, D)), _full_spec((D, ncls)), _full_spec((1, ncls))],
        out_specs=_row_spec(ncls),
        out_shape=jax.ShapeDtypeStruct((N, ncls), jnp.float32),
    )(p2, g2, dis, b2r, Wfc, bfcr)

    return out


# speedup vs baseline: 38.4149x; 1.1109x over previous
import functools

import jax
import jax.numpy as jnp
from jax import lax
from jax.experimental import pallas as pl
from jax.experimental.pallas import tpu as pltpu
from jax.experimental.pallas import tpu_sc as plsc

N = 10000
NPAD = 10240
E = 320000
D = 128
NC = 2
NS = 16
NW = NC * NS
EPW = E // NW
K = 80
NCHUNK = EPW // K
RPT = 624
RTAIL = N - NS * RPT
NROW = NPAD // 128

_mesh = plsc.VectorSubcoreMesh(core_axis_name="c", subcore_axis_name="s")


def _copy_rows(src_ref, dst_ref, s):
    pltpu.sync_copy(src_ref.at[pl.ds(s * RPT, RPT)], dst_ref.at[pl.ds(s * RPT, RPT)])

    @pl.when(s == NS - 1)
    def _():
        pltpu.sync_copy(src_ref.at[pl.ds(NS * RPT, RTAIL)],
                        dst_ref.at[pl.ds(NS * RPT, RTAIL)])


@functools.partial(
    pl.kernel,
    out_type=jax.ShapeDtypeStruct((NC, NROW, 128), jnp.float32),
    mesh=_mesh,
    scratch_types=[
        pltpu.VMEM_SHARED((NROW, 128), jnp.float32),
        pltpu.VMEM((NROW, 128), jnp.float32),
        pltpu.VMEM((EPW,), jnp.int32),
        pltpu.VMEM((NROW,), jnp.int32),
    ],
    compiler_params=pltpu.CompilerParams(needs_layout_passes=False),
)
def _sc_degree(ei_hbm, out_hbm, acc, hist_v, dst_v, iota_v):
    c = lax.axis_index("c")
    s = lax.axis_index("s")
    wid = c * NS + s
    zeros16 = jnp.zeros((16,), jnp.float32)
    ones16 = jnp.ones((16,), jnp.float32)

    for j in range(NROW):
        for q in range(8):
            hist_v[j, pl.ds(q * 16, 16)] = zeros16

    @pl.when(s == 0)
    def _():
        pltpu.sync_copy(hist_v, acc)

    def fill_iota(j, carry):
        iota_v[pl.ds(j * 16, 16)] = lax.iota(jnp.int32, 16) + j * 16
        return carry

    lax.fori_loop(0, NROW // 16, fill_iota, 0)
    pltpu.sync_copy(ei_hbm.at[pl.ds(E + wid * EPW, EPW)], dst_v)
    plsc.subcore_barrier()

    def step(j, carry):
        vec = dst_v[pl.ds(j * 16, 16)]
        hi = lax.shift_right_logical(vec, 7)
        lo = lax.bitwise_and(vec, 127)
        plsc.addupdate_scatter(hist_v, [hi, lo], ones16)
        return carry

    lax.fori_loop(0, EPW // 16, step, 0)
    pltpu.sync_copy(hist_v, acc.at[iota_v], add=True)
    plsc.subcore_barrier()

    @pl.when(s == 0)
    def _():
        pltpu.sync_copy(acc, out_hbm.at[c])


NBUF = 4
GMAX = (NCHUNK - 1) // NBUF


@functools.partial(
    pl.kernel,
    out_type=jax.ShapeDtypeStruct((NC, N, D), jnp.float32),
    mesh=_mesh,
    scratch_types=[
        pltpu.VMEM_SHARED((N, D), jnp.float32),
        pltpu.VMEM((NBUF, K), jnp.int32),
        pltpu.VMEM((NBUF, K), jnp.int32),
        pltpu.VMEM((NBUF, K, D), jnp.float32),
        pltpu.SemaphoreType.DMA((NBUF,)),
        pltpu.SemaphoreType.DMA((NBUF,)),
        pltpu.SemaphoreType.DMA((NBUF,)),
    ],
)
def _sc_scatter(g_hbm, ei_hbm, zeros_hbm, out_hbm,
                acc, src_v, dst_v, rows_v, semg, semis, semid):
    c = lax.axis_index("c")
    s = lax.axis_index("s")
    wid = c * NS + s
    _copy_rows(zeros_hbm, acc, s)

    def fire_src(b, i):
        pltpu.async_copy(ei_hbm.at[pl.ds(wid * EPW + i * K, K)],
                         src_v.at[b], semis.at[b])

    def fire_dst(b, i):
        pltpu.async_copy(ei_hbm.at[pl.ds(E + wid * EPW + i * K, K)],
                         dst_v.at[b], semid.at[b])

    def fire_gather(b):
        pltpu.async_copy(g_hbm.at[src_v.at[b]], rows_v.at[b], semg.at[b])

    for b in range(NBUF):
        fire_src(b, b)
        fire_dst(b, b)
    for b in range(NBUF):
        pltpu.make_async_copy(ei_hbm.at[pl.ds(0, K)], src_v.at[b],
                              semis.at[b]).wait()
        fire_gather(b)

    plsc.subcore_barrier()

    def group(g, carry):
        has_next = g < GMAX - 1
        for b in range(NBUF):
            i = g * NBUF + b
            pltpu.make_async_copy(g_hbm.at[src_v.at[b]], rows_v.at[b],
                                  semg.at[b]).wait()

            @pl.when(has_next)
            def _():
                fire_src(b, i + NBUF)
            pltpu.make_async_copy(ei_hbm.at[pl.ds(0, K)], dst_v.at[b],
                                  semid.at[b]).wait()
            pltpu.sync_copy(rows_v.at[b], acc.at[dst_v.at[b]], add=True)

            @pl.when(has_next)
            def _():
                fire_dst(b, i + NBUF)
                pltpu.make_async_copy(ei_hbm.at[pl.ds(0, K)], src_v.at[b],
                                      semis.at[b]).wait()
                fire_gather(b)
        return carry

    lax.fori_loop(0, GMAX, group, 0)

    tail = NCHUNK - 1
    pltpu.sync_copy(ei_hbm.at[pl.ds(wid * EPW + tail * K, K)], src_v.at[0])
    pltpu.sync_copy(ei_hbm.at[pl.ds(E + wid * EPW + tail * K, K)], dst_v.at[0])
    pltpu.async_copy(g_hbm.at[src_v.at[0]], rows_v.at[0], semg.at[0])
    pltpu.make_async_copy(g_hbm.at[src_v.at[0]], rows_v.at[0], semg.at[0]).wait()
    pltpu.sync_copy(rows_v.at[0], acc.at[dst_v.at[0]], add=True)

    plsc.subcore_barrier()
    _copy_rows(acc, out_hbm.at[c], s)


RB = 2000


def _tc_dis_body(deg_ref, dis_ref):
    deg = deg_ref[0] + deg_ref[1] + 1.0
    dis_ref[...] = 1.0 / jnp.sqrt(deg)


def _tc_g1_body(x_ref, w1_ref, dis_ref, g1_ref):
    h = jnp.dot(x_ref[...], w1_ref[...], preferred_element_type=jnp.float32)
    g1_ref[...] = dis_ref[...] * h


def _tc_layer2_body(p_ref, g1_ref, dis_ref, b1_ref, w2_ref, g2_ref):
    dis = dis_ref[...]
    p = p_ref[...]
    s = p[0] + p[1] + g1_ref[...]
    h1 = jnp.maximum(dis * s + b1_ref[...], 0.0)
    h2 = jnp.dot(h1, w2_ref[...], preferred_element_type=jnp.float32)
    g2_ref[...] = dis * h2


def _tc_final_body(q_ref, g2_ref, dis_ref, b2_ref, wfc_ref, bfc_ref, out_ref):
    dis = dis_ref[...]
    q = q_ref[...]
    s = q[0] + q[1] + g2_ref[...]
    h2 = dis * s + b2_ref[...]
    out_ref[...] = jnp.dot(h2, wfc_ref[...], preferred_element_type=jnp.float32) + bfc_ref[...]


def _row_spec(cols):
    return pl.BlockSpec((RB, cols), lambda i: (i, 0))


def _full_spec(shape):
    return pl.BlockSpec(shape, lambda i: tuple(0 for _ in shape))


def _pair_spec(cols):
    return pl.BlockSpec((NC, RB, cols), lambda i: (0, i, 0))


def kernel(x, edge_index, W1, b1, W2, b2, Wfc, bfc):
    ei = edge_index.astype(jnp.int32).reshape(2 * E)
    zerosD = jnp.zeros((N, D), jnp.float32)
    b1r = b1.reshape(1, D)
    b2r = b2.reshape(1, D)
    bfcr = bfc.reshape(1, -1)
    ncls = Wfc.shape[1]

    degp = _sc_degree(ei)

    dis2d = pl.pallas_call(
        _tc_dis_body,
        in_specs=[pl.BlockSpec((NC, NROW, 128), lambda: (0, 0, 0))],
        out_specs=pl.BlockSpec((NROW, 128), lambda: (0, 0)),
        out_shape=jax.ShapeDtypeStruct((NROW, 128), jnp.float32),
    )(degp)
    dis = dis2d.reshape(NPAD, 1)[:N]

    g1 = pl.pallas_call(
        _tc_g1_body,
        grid=(N // RB,),
        in_specs=[_row_spec(D), _full_spec((D, D)), _row_spec(1)],
        out_specs=_row_spec(D),
        out_shape=jax.ShapeDtypeStruct((N, D), jnp.float32),
    )(x, W1, dis)

    p1 = _sc_scatter(g1, ei, zerosD)

    g2 = pl.pallas_call(
        _tc_layer2_body,
        grid=(N // RB,),
        in_specs=[_pair_spec(D), _row_spec(D), _row_spec(1),
                  _full_spec((1, D)), _full_spec((D, D))],
        out_specs=_row_spec(D),
        out_shape=jax.ShapeDtypeStruct((N, D), jnp.float32),
    )(p1, g1, dis, b1r, W2)

    p2 = _sc_scatter(g2, ei, zerosD)

    out = pl.pallas_call(
        _tc_final_body,
        grid=(N // RB,),
        in_specs=[_pair_spec(D), _row_spec(D), _row_spec(1),
                  _full_spec((1, D)), _full_spec((D, ncls)), _full_spec((1, ncls))],
        out_specs=_row_spec(ncls),
        out_shape=jax.ShapeDtypeStruct((N, ncls), jnp.float32),
    )(p2, g2, dis, b2r, Wfc, bfcr)

    return out


# speedup vs baseline: 38.9510x; 1.0140x over previous
import functools

import jax
import jax.numpy as jnp
from jax import lax
from jax.experimental import pallas as pl
from jax.experimental.pallas import tpu as pltpu
from jax.experimental.pallas import tpu_sc as plsc

N = 10000
NPAD = 10240
E = 320000
D = 128
NC = 2
NS = 16
NW = NC * NS
EPW = E // NW
K = 80
NCHUNK = EPW // K
RPT = 624
RTAIL = N - NS * RPT
NROW = NPAD // 128

_mesh = plsc.VectorSubcoreMesh(core_axis_name="c", subcore_axis_name="s")


def _copy_rows(src_ref, dst_ref, s):
    pltpu.sync_copy(src_ref.at[pl.ds(s * RPT, RPT)], dst_ref.at[pl.ds(s * RPT, RPT)])

    @pl.when(s == NS - 1)
    def _():
        pltpu.sync_copy(src_ref.at[pl.ds(NS * RPT, RTAIL)],
                        dst_ref.at[pl.ds(NS * RPT, RTAIL)])


@functools.partial(
    pl.kernel,
    out_type=jax.ShapeDtypeStruct((NC, NROW, 128), jnp.float32),
    mesh=_mesh,
    scratch_types=[
        pltpu.VMEM_SHARED((NROW, 128), jnp.float32),
        pltpu.VMEM((NROW, 128), jnp.float32),
        pltpu.VMEM((EPW,), jnp.int32),
        pltpu.VMEM((NROW,), jnp.int32),
    ],
    compiler_params=pltpu.CompilerParams(needs_layout_passes=False),
)
def _sc_degree(ei_hbm, out_hbm, acc, hist_v, dst_v, iota_v):
    c = lax.axis_index("c")
    s = lax.axis_index("s")
    wid = c * NS + s
    zeros16 = jnp.zeros((16,), jnp.float32)
    ones16 = jnp.ones((16,), jnp.float32)

    for j in range(NROW):
        for q in range(8):
            hist_v[j, pl.ds(q * 16, 16)] = zeros16

    @pl.when(s == 0)
    def _():
        pltpu.sync_copy(hist_v, acc)

    def fill_iota(j, carry):
        iota_v[pl.ds(j * 16, 16)] = lax.iota(jnp.int32, 16) + j * 16
        return carry

    lax.fori_loop(0, NROW // 16, fill_iota, 0)
    pltpu.sync_copy(ei_hbm.at[pl.ds(E + wid * EPW, EPW)], dst_v)
    plsc.subcore_barrier()

    def step(j, carry):
        vec = dst_v[pl.ds(j * 16, 16)]
        hi = lax.shift_right_logical(vec, 7)
        lo = lax.bitwise_and(vec, 127)
        plsc.addupdate_scatter(hist_v, [hi, lo], ones16)
        return carry

    lax.fori_loop(0, EPW // 16, step, 0)
    pltpu.sync_copy(hist_v, acc.at[iota_v], add=True)
    plsc.subcore_barrier()

    @pl.when(s == 0)
    def _():
        pltpu.sync_copy(acc, out_hbm.at[c])


NBUF = 4
GMAX = (NCHUNK - 1) // NBUF


@functools.partial(
    pl.kernel,
    out_type=jax.ShapeDtypeStruct((NC, N, D), jnp.float32),
    mesh=_mesh,
    scratch_types=[
        pltpu.VMEM_SHARED((N, D), jnp.float32),
        pltpu.VMEM((NBUF, K), jnp.int32),
        pltpu.VMEM((NBUF, K), jnp.int32),
        pltpu.VMEM((NBUF, K, D), jnp.float32),
        pltpu.SemaphoreType.DMA((NBUF,)),
        pltpu.SemaphoreType.DMA((NBUF,)),
        pltpu.SemaphoreType.DMA((NBUF,)),
    ],
)
def _sc_scatter(g_hbm, ei_hbm, out_hbm,
                acc, src_v, dst_v, rows_v, semg, semis, semid):
    c = lax.axis_index("c")
    s = lax.axis_index("s")
    wid = c * NS + s
    _copy_rows(g_hbm, acc, s)

    def fire_src(b, i):
        pltpu.async_copy(ei_hbm.at[pl.ds(wid * EPW + i * K, K)],
                         src_v.at[b], semis.at[b])

    def fire_dst(b, i):
        pltpu.async_copy(ei_hbm.at[pl.ds(E + wid * EPW + i * K, K)],
                         dst_v.at[b], semid.at[b])

    def fire_gather(b):
        pltpu.async_copy(g_hbm.at[src_v.at[b]], rows_v.at[b], semg.at[b])

    for b in range(NBUF):
        fire_src(b, b)
        fire_dst(b, b)
    for b in range(NBUF):
        pltpu.make_async_copy(ei_hbm.at[pl.ds(0, K)], src_v.at[b],
                              semis.at[b]).wait()
        fire_gather(b)

    plsc.subcore_barrier()

    def group(g, carry):
        has_next = g < GMAX - 1
        for b in range(NBUF):
            i = g * NBUF + b
            pltpu.make_async_copy(g_hbm.at[src_v.at[b]], rows_v.at[b],
                                  semg.at[b]).wait()

            @pl.when(has_next)
            def _():
                fire_src(b, i + NBUF)
            pltpu.make_async_copy(ei_hbm.at[pl.ds(0, K)], dst_v.at[b],
                                  semid.at[b]).wait()
            pltpu.sync_copy(rows_v.at[b], acc.at[dst_v.at[b]], add=True)

            @pl.when(has_next)
            def _():
                fire_dst(b, i + NBUF)
                pltpu.make_async_copy(ei_hbm.at[pl.ds(0, K)], src_v.at[b],
                                      semis.at[b]).wait()
                fire_gather(b)
        return carry

    lax.fori_loop(0, GMAX, group, 0)

    tail = NCHUNK - 1
    pltpu.sync_copy(ei_hbm.at[pl.ds(wid * EPW + tail * K, K)], src_v.at[0])
    pltpu.sync_copy(ei_hbm.at[pl.ds(E + wid * EPW + tail * K, K)], dst_v.at[0])
    pltpu.async_copy(g_hbm.at[src_v.at[0]], rows_v.at[0], semg.at[0])
    pltpu.make_async_copy(g_hbm.at[src_v.at[0]], rows_v.at[0], semg.at[0]).wait()
    pltpu.sync_copy(rows_v.at[0], acc.at[dst_v.at[0]], add=True)

    plsc.subcore_barrier()
    _copy_rows(acc, out_hbm.at[c], s)


RB = 2000


def _tc_dis_body(deg_ref, dis_ref):
    deg = deg_ref[0] + deg_ref[1] + 1.0
    dis_ref[...] = 1.0 / jnp.sqrt(deg)


def _tc_g1_body(x_ref, w1_ref, dis_ref, g1_ref):
    h = jnp.dot(x_ref[...], w1_ref[...], preferred_element_type=jnp.float32)
    g1_ref[...] = dis_ref[...] * h


def _tc_layer2_body(p_ref, g1_ref, dis_ref, b1_ref, w2_ref, g2_ref):
    dis = dis_ref[...]
    p = p_ref[...]
    s = p[0] + p[1] - g1_ref[...]
    h1 = jnp.maximum(dis * s + b1_ref[...], 0.0)
    h2 = jnp.dot(h1, w2_ref[...], preferred_element_type=jnp.float32)
    g2_ref[...] = dis * h2


def _tc_final_body(q_ref, g2_ref, dis_ref, b2_ref, wfc_ref, bfc_ref, out_ref):
    dis = dis_ref[...]
    q = q_ref[...]
    s = q[0] + q[1] - g2_ref[...]
    h2 = dis * s + b2_ref[...]
    out_ref[...] = jnp.dot(h2, wfc_ref[...], preferred_element_type=jnp.float32) + bfc_ref[...]


def _row_spec(cols):
    return pl.BlockSpec((RB, cols), lambda i: (i, 0))


def _full_spec(shape):
    return pl.BlockSpec(shape, lambda i: tuple(0 for _ in shape))


def _pair_spec(cols):
    return pl.BlockSpec((NC, RB, cols), lambda i: (0, i, 0))


def kernel(x, edge_index, W1, b1, W2, b2, Wfc, bfc):
    ei = edge_index.astype(jnp.int32).reshape(2 * E)
    b1r = b1.reshape(1, D)
    b2r = b2.reshape(1, D)
    bfcr = bfc.reshape(1, -1)
    ncls = Wfc.shape[1]

    degp = _sc_degree(ei)

    dis2d = pl.pallas_call(
        _tc_dis_body,
        in_specs=[pl.BlockSpec((NC, NROW, 128), lambda: (0, 0, 0))],
        out_specs=pl.BlockSpec((NROW, 128), lambda: (0, 0)),
        out_shape=jax.ShapeDtypeStruct((NROW, 128), jnp.float32),
    )(degp)
    dis = dis2d.reshape(NPAD, 1)

    g1 = pl.pallas_call(
        _tc_g1_body,
        grid=(N // RB,),
        in_specs=[_row_spec(D), _full_spec((D, D)), _row_spec(1)],
        out_specs=_row_spec(D),
        out_shape=jax.ShapeDtypeStruct((N, D), jnp.float32),
    )(x, W1, dis)

    p1 = _sc_scatter(g1, ei)

    g2 = pl.pallas_call(
        _tc_layer2_body,
        grid=(N // RB,),
        in_specs=[_pair_spec(D), _row_spec(D), _row_spec(1),
                  _full_spec((1, D)), _full_spec((D, D))],
        out_specs=_row_spec(D),
        out_shape=jax.ShapeDtypeStruct((N, D), jnp.float32),
    )(p1, g1, dis, b1r, W2)

    p2 = _sc_scatter(g2, ei)

    out = pl.pallas_call(
        _tc_final_body,
        grid=(N // RB,),
        in_specs=[_pair_spec(D), _row_spec(D), _row_spec(1),
                  _full_spec((1, D)), _full_spec((D, ncls)), _full_spec((1, ncls))],
        out_specs=_row_spec(ncls),
        out_shape=jax.ShapeDtypeStruct((N, ncls), jnp.float32),
    )(p2, g2, dis, b2r, Wfc, bfcr)

    return out
